# Initial kernel scaffold; baseline (speedup 1.0000x reference)
import jax
import jax.numpy as jnp
from jax.experimental import pallas as pl


def kernel(mm_0_a, mm_0_b, eb_input, eb_offset, W0, W1, W2):
    raise NotImplementedError("write your pallas kernel here")


# speedup vs baseline: 169.6520x; 169.6520x over previous
import functools

import jax
import jax.numpy as jnp
from jax import lax
from jax.experimental import pallas as pl
from jax.experimental.pallas import tpu as pltpu
from jax.experimental.pallas import tpu_sc as plsc

L = 16
NW = 32
N_IN = 16384
CPT = N_IN // NW
NV = 5
D = 14

_mesh = plsc.VectorSubcoreMesh(core_axis_name="c", subcore_axis_name="s")


@functools.partial(
    pl.kernel,
    mesh=_mesh,
    out_type=jax.ShapeDtypeStruct((NW, L), jnp.float32),
    scratch_types=[
        pltpu.VMEM((CPT,), jnp.int32),
        pltpu.VMEM((L,), jnp.float32),
    ],
)
def _sc_histogram(e_hbm, out_hbm, idx_v, part_v):
    c = lax.axis_index("c")
    s = lax.axis_index("s")
    wid = s * 2 + c
    lane = lax.broadcasted_iota(jnp.int32, (L,), 0)

    def lane_sum(x):
        for sh in (8, 4, 2, 1):
            x = x + x.at[lane ^ sh].get(mode="promise_in_bounds",
                                        unique_indices=True)
        return x

    pltpu.sync_copy(e_hbm.at[pl.ds(wid * CPT, CPT)], idx_v)
    acc = [jnp.zeros((L,), jnp.float32) for _ in range(NV)]
    for k in range(CPT // L):
        x = jnp.clip(idx_v[pl.ds(k * L, L)], 0, NV - 1)
        for v in range(NV):
            acc[v] = acc[v] + jnp.where(x == v, 1.0, 0.0)
    part = jnp.zeros((L,), jnp.float32)
    for v in range(NV):
        part = jnp.where(lane == v, lane_sum(acc[v]), part)
    part_v[...] = part
    pltpu.sync_copy(part_v, out_hbm.at[wid])


def _tc_tail(part_ref, a_ref, b_ref, w0_ref, w1_ref, w2_ref, out_ref):
    counts = jnp.sum(part_ref[...], axis=0, keepdims=True)
    c5 = counts[:, :NV]
    e0 = c5 @ jnp.sum(w0_ref[...], axis=0)
    e1 = c5 @ jnp.sum(w1_ref[...], axis=0)
    e2 = c5 @ jnp.sum(w2_ref[...], axis=0)
    mm0 = a_ref[...] @ b_ref[...]
    s1 = jnp.sum(e1[0, :] * mm0[:, 0])
    s2 = jnp.sum(e2 * e0)
    out_ref[...] = jnp.full((1, 1), s1 * s2, jnp.float32)


_tc_tail_call = pl.pallas_call(
    _tc_tail,
    out_shape=jax.ShapeDtypeStruct((1, 1), jnp.float32),
)


def kernel(mm_0_a, mm_0_b, eb_input, eb_offset, W0, W1, W2):
    del eb_offset
    part = _sc_histogram(eb_input)
    return _tc_tail_call(part, mm_0_a, mm_0_b, W0, W1, W2)
